# baseline (speedup 1.0000x reference)
import functools

import jax
import jax.numpy as jnp
from jax import lax
from jax.experimental import pallas as pl
from jax.experimental.pallas import tpu as pltpu
from jax.experimental.pallas import tpu_sc as plsc

_L = 16


def _build_sc_call(batch, height, width, num_feats):
    hw = height * width
    two_d = 2 * num_feats
    n_workers = 32
    chans_per_w = two_d // n_workers
    blk_len = chans_per_w * hw
    out_len = batch * two_d * hw

    mesh = plsc.VectorSubcoreMesh(core_axis_name="c", subcore_axis_name="s")

    @functools.partial(
        pl.kernel,
        mesh=mesh,
        out_type=jax.ShapeDtypeStruct((out_len,), jnp.float32),
        scratch_types=[
            pltpu.VMEM((width * num_feats,), jnp.float32),
            pltpu.VMEM((height * num_feats,), jnp.float32),
            pltpu.VMEM((blk_len,), jnp.float32),
            pltpu.SemaphoreType.DMA,
        ],
        compiler_params=pltpu.CompilerParams(needs_layout_passes=False),
    )
    def _k(row_hbm, col_hbm, out_hbm, colv, rowv, blk, sem):
        wid = lax.axis_index("s") * 2 + lax.axis_index("c")
        is_col = wid < (n_workers // 2)

        pltpu.sync_copy(col_hbm.at[pl.ds(0, width * num_feats)], colv)
        pltpu.sync_copy(row_hbm.at[pl.ds(0, height * num_feats)], rowv)

        iot = lax.iota(jnp.int32, _L)
        zeros = jnp.zeros((_L,), jnp.int32)

        @pl.when(is_col)
        def _():
            for c in range(chans_per_w):
                cvec = zeros + (wid * chans_per_w + c)
                v0 = plsc.load_gather(colv, [iot * num_feats + cvec])
                v1 = plsc.load_gather(colv, [(iot + _L) * num_feats + cvec])

                def body(h, _, c=c, v0=v0, v1=v1):
                    base = c * hw + h * width
                    blk[pl.ds(base, _L)] = v0
                    blk[pl.ds(base + _L, _L)] = v1
                    return 0

                lax.fori_loop(0, height, body, 0)

        @pl.when(jnp.logical_not(is_col))
        def _():
            for c in range(chans_per_w):
                cvec = zeros + (wid * chans_per_w + c - num_feats)

                def body(h, _, c=c, cvec=cvec):
                    v = plsc.load_gather(rowv, [h * num_feats + cvec])
                    base = c * hw + h * width
                    blk[pl.ds(base, _L)] = v
                    blk[pl.ds(base + _L, _L)] = v
                    return 0

                lax.fori_loop(0, height, body, 0)

        copies = []
        for b in range(batch):
            off = b * (two_d * hw) + wid * blk_len
            copies.append(pltpu.async_copy(blk, out_hbm.at[pl.ds(off, blk_len)], sem))
        for cp in copies:
            cp.wait()

    return _k


def kernel(pixel_values, pixel_mask, row_embed, col_embed):
    batch = pixel_values.shape[0]
    height, width = pixel_values.shape[-2:]
    num_feats = row_embed.shape[1]
    call = _build_sc_call(batch, height, width, num_feats)
    flat = call(row_embed.reshape(-1), col_embed.reshape(-1))
    return flat.reshape(batch, 2 * num_feats, height, width)


# speedup vs baseline: 1.1304x; 1.1304x over previous
import functools

import jax
import jax.numpy as jnp
from jax import lax
from jax.experimental import pallas as pl
from jax.experimental.pallas import tpu as pltpu
from jax.experimental.pallas import tpu_sc as plsc

_L = 16


def _build_sc_call(batch, height, width, num_feats):
    hw = height * width
    two_d = 2 * num_feats
    n_workers = 32
    chans_per_w = two_d // n_workers

    mesh = plsc.VectorSubcoreMesh(core_axis_name="c", subcore_axis_name="s")

    @functools.partial(
        pl.kernel,
        mesh=mesh,
        out_type=jax.ShapeDtypeStruct((batch, two_d, height, width), jnp.float32),
        scratch_types=[
            pltpu.VMEM((width * num_feats,), jnp.float32),
            pltpu.VMEM((height * num_feats,), jnp.float32),
            pltpu.VMEM((chans_per_w, height, width), jnp.float32),
            pltpu.SemaphoreType.DMA,
        ],
        compiler_params=pltpu.CompilerParams(needs_layout_passes=False),
    )
    def _k(row_hbm, col_hbm, out_hbm, colv, rowv, blk, sem):
        wid = lax.axis_index("s") * 2 + lax.axis_index("c")
        is_col = wid < (n_workers // 2)

        pltpu.sync_copy(col_hbm.at[pl.ds(0, width * num_feats)], colv)
        pltpu.sync_copy(row_hbm.at[pl.ds(0, height * num_feats)], rowv)

        iot = lax.iota(jnp.int32, _L)
        zeros = jnp.zeros((_L,), jnp.int32)

        @pl.when(is_col)
        def _():
            for c in range(chans_per_w):
                cvec = zeros + (wid * chans_per_w + c)
                v0 = plsc.load_gather(colv, [iot * num_feats + cvec])
                v1 = plsc.load_gather(colv, [(iot + _L) * num_feats + cvec])

                def body(h, _, c=c, v0=v0, v1=v1):
                    blk[c, h, pl.ds(0, _L)] = v0
                    blk[c, h, pl.ds(_L, _L)] = v1
                    return 0

                lax.fori_loop(0, height, body, 0)

        @pl.when(jnp.logical_not(is_col))
        def _():
            for c in range(chans_per_w):
                cvec = zeros + (wid * chans_per_w + c - num_feats)

                def body(h, _, c=c, cvec=cvec):
                    v = plsc.load_gather(rowv, [h * num_feats + cvec])
                    blk[c, h, pl.ds(0, _L)] = v
                    blk[c, h, pl.ds(_L, _L)] = v
                    return 0

                lax.fori_loop(0, height, body, 0)

        copies = []
        for b in range(batch):
            copies.append(
                pltpu.async_copy(
                    blk,
                    out_hbm.at[b, pl.ds(wid * chans_per_w, chans_per_w)],
                    sem,
                )
            )
        for cp in copies:
            cp.wait()

    return _k


def kernel(pixel_values, pixel_mask, row_embed, col_embed):
    batch = pixel_values.shape[0]
    height, width = pixel_values.shape[-2:]
    num_feats = row_embed.shape[1]
    call = _build_sc_call(batch, height, width, num_feats)
    return call(row_embed.reshape(-1), col_embed.reshape(-1))
